# baseline (speedup 1.0000x reference)
import functools

import jax
import jax.numpy as jnp
from jax import lax
from jax.experimental import pallas as pl
from jax.experimental.pallas import tpu as pltpu
from jax.experimental.pallas import tpu_sc as plsc

BATCH = 4096
SEQ = 200
SLC0 = 104
SLC1 = SEQ - SLC0
EMB = 32
HID = 128
OUT = 64
VOCAB = 1000000
PHYS_ROWS = VOCAB // 4
PHYS_W = 128

NUM_WORKERS = 32
ROWS_PER_W = BATCH // NUM_WORKERS
ROWS_PER_G = 2
GROUPS = ROWS_PER_W // ROWS_PER_G

CHUNKS0 = ((0, 0), (16, 0), (32, 0), (48, 0), (64, 0), (80, 0), (88, 8))
CHUNKS1 = ((0, 0), (16, 0), (32, 0), (48, 0), (64, 0), (80, 0))
ROW_CHUNKS = (0, 16, 32, 48, 64, 80, 96, 112, 128, 144, 160, 176, 184)


def _pool_body(table_hbm, idx_hbm, out_hbm, idx_v, phys_v, b0, b1, b2, b3,
               outs_v, s0, s1, s2, s3):
    bufs = (b0, b1, b2, b3)
    sems = (s0, s1, s2, s3)
    wid = lax.axis_index("s") * 2 + lax.axis_index("c")
    base_row = wid * ROWS_PER_W

    pltpu.sync_copy(idx_hbm.at[pl.ds(base_row, ROWS_PER_W)], idx_v)

    def prep_row(h, carry):
        for o in ROW_CHUNKS:
            v = idx_v[h, pl.ds(o, 16)]
            v = jnp.where(v == VOCAB, 0, v)
            idx_v[h, pl.ds(o, 16)] = v
            phys_v[h, pl.ds(o, 16)] = lax.shift_right_logical(v, 2)
        return carry

    lax.fori_loop(0, ROWS_PER_W, prep_row, 0)

    def group(g, carry):
        cps = []
        for k in range(2 * ROWS_PER_G):
            row = ROWS_PER_G * g + k // 2
            sl = pl.ds(0, SLC0) if k % 2 == 0 else pl.ds(SLC0, SLC1)
            cps.append(pltpu.async_copy(table_hbm.at[phys_v.at[row, sl]],
                                        bufs[k], sems[k]))
        for r in range(ROWS_PER_G):
            acc_lo = jnp.zeros((16,), jnp.float32)
            acc_hi = jnp.zeros((16,), jnp.float32)
            row = ROWS_PER_G * g + r
            for k in (2 * r, 2 * r + 1):
                cps[k].wait()
                buf = bufs[k]
                base = 0 if k % 2 == 0 else SLC0
                for o, j0 in (CHUNKS0 if k % 2 == 0 else CHUNKS1):
                    sv = idx_v[row, pl.ds(base + o, 16)]
                    offs = lax.bitwise_and(sv, 3) * 32
                    for j in range(j0, 16):
                        s = o + j
                        off = pl.multiple_of(offs[j], 32)
                        acc_lo = acc_lo + buf[s, pl.ds(off, 16)]
                        acc_hi = acc_hi + buf[s, pl.ds(off + 16, 16)]
            outs_v[row, 0:16] = acc_lo
            outs_v[row, 16:32] = acc_hi
        return carry

    lax.fori_loop(0, GROUPS, group, 0)
    pltpu.sync_copy(outs_v, out_hbm.at[pl.ds(base_row, ROWS_PER_W)])


_pooled_sum = functools.partial(
    pl.kernel,
    mesh=plsc.VectorSubcoreMesh(core_axis_name="c", subcore_axis_name="s"),
    compiler_params=pltpu.CompilerParams(use_tc_tiling_on_sc=False),
    out_type=jax.ShapeDtypeStruct((BATCH, EMB), jnp.float32),
    scratch_types=[
        pltpu.VMEM((ROWS_PER_W, SEQ), jnp.int32),
        pltpu.VMEM((ROWS_PER_W, SEQ), jnp.int32),
        pltpu.VMEM((SLC0, PHYS_W), jnp.float32),
        pltpu.VMEM((SLC1, PHYS_W), jnp.float32),
        pltpu.VMEM((SLC0, PHYS_W), jnp.float32),
        pltpu.VMEM((SLC1, PHYS_W), jnp.float32),
        pltpu.VMEM((ROWS_PER_W, EMB), jnp.float32),
        pltpu.SemaphoreType.DMA,
        pltpu.SemaphoreType.DMA,
        pltpu.SemaphoreType.DMA,
        pltpu.SemaphoreType.DMA,
    ],
)(_pool_body)


def _mlp_body(pooled_ref, idx_ref, t0_ref, tl_ref, w1_ref, bb1_ref, w2_ref,
              bb2_ref, out_ref):
    pooled = pooled_ref[...]
    idx = idx_ref[...]
    c0 = jnp.sum((idx == 0).astype(jnp.float32), axis=1, keepdims=True)
    cl = jnp.sum((idx == VOCAB).astype(jnp.float32), axis=1, keepdims=True)
    x = (pooled - (c0 + cl) * t0_ref[...] + cl * tl_ref[...]) * (1.0 / SEQ)
    h = jnp.dot(x, w1_ref[...], preferred_element_type=jnp.float32,
                precision=lax.Precision.HIGHEST) + bb1_ref[...]
    z = jnp.dot(h, w2_ref[...], preferred_element_type=jnp.float32,
                precision=lax.Precision.HIGHEST) + bb2_ref[...]
    z = z - jnp.max(z, axis=1, keepdims=True)
    e = jnp.exp(z)
    out_ref[...] = e / jnp.sum(e, axis=1, keepdims=True)


def _mlp_call(pooled, idx, t0, tl, w1, bb1, w2, bb2):
    bt = 512
    grid = (BATCH // bt,)
    return pl.pallas_call(
        _mlp_body,
        out_shape=jax.ShapeDtypeStruct((BATCH, OUT), jnp.float32),
        grid=grid,
        in_specs=[
            pl.BlockSpec((bt, EMB), lambda i: (i, 0)),
            pl.BlockSpec((bt, SEQ), lambda i: (i, 0)),
            pl.BlockSpec((1, EMB), lambda i: (0, 0)),
            pl.BlockSpec((1, EMB), lambda i: (0, 0)),
            pl.BlockSpec((EMB, HID), lambda i: (0, 0)),
            pl.BlockSpec((1, HID), lambda i: (0, 0)),
            pl.BlockSpec((HID, OUT), lambda i: (0, 0)),
            pl.BlockSpec((1, OUT), lambda i: (0, 0)),
        ],
        out_specs=pl.BlockSpec((bt, OUT), lambda i: (i, 0)),
    )(pooled, idx, t0, tl, w1, bb1, w2, bb2)


def kernel(inputs, table, W1, b1, W2, b2):
    idx = inputs.astype(jnp.int32)
    table128 = table[:VOCAB].reshape(PHYS_ROWS, PHYS_W)
    pooled = _pooled_sum(table128, idx)
    t0 = table[0:1]
    tl = table[VOCAB:VOCAB + 1]
    return _mlp_call(pooled, idx, t0, tl, W1, b1.reshape(1, HID), W2,
                     b2.reshape(1, OUT))


# speedup vs baseline: 1.1682x; 1.1682x over previous
import functools

import jax
import jax.numpy as jnp
from jax import lax
from jax.experimental import pallas as pl
from jax.experimental.pallas import tpu as pltpu
from jax.experimental.pallas import tpu_sc as plsc

BATCH = 4096
SEQ = 200
SLC0 = 104
SLC1 = SEQ - SLC0
EMB = 32
HID = 128
OUT = 64

NUM_WORKERS = 32
ROWS_PER_W = BATCH // NUM_WORKERS
ROWS_PER_G = 8
GROUPS = ROWS_PER_W // ROWS_PER_G


def _pool_body(table_hbm, idx_hbm, out_hbm, idx_v, b0, b1, b2, b3, b4, b5,
               b6, b7, b8, b9, b10, b11, b12, b13, b14, b15, outs_v,
               s0, s1, s2, s3, s4, s5, s6, s7, s8, s9, s10, s11, s12, s13,
               s14, s15):
    bufs = (b0, b1, b2, b3, b4, b5, b6, b7, b8, b9, b10, b11, b12, b13,
            b14, b15)
    sems = (s0, s1, s2, s3, s4, s5, s6, s7, s8, s9, s10, s11, s12, s13,
            s14, s15)
    wid = lax.axis_index("s") * 2 + lax.axis_index("c")
    base_row = wid * ROWS_PER_W

    pltpu.sync_copy(idx_hbm.at[pl.ds(base_row, ROWS_PER_W)], idx_v)

    def group(g, carry):
        cps = []
        for k in range(2 * ROWS_PER_G):
            row = ROWS_PER_G * g + k // 2
            sl = pl.ds(0, SLC0) if k % 2 == 0 else pl.ds(SLC0, SLC1)
            cps.append(pltpu.async_copy(table_hbm.at[idx_v.at[row, sl]],
                                        bufs[k], sems[k]))
        for r in range(ROWS_PER_G):
            acc_lo = jnp.zeros((16,), jnp.float32)
            acc_hi = jnp.zeros((16,), jnp.float32)
            for k in (2 * r, 2 * r + 1):
                cps[k].wait()
                buf = bufs[k]
                for s in range(SLC0 if k % 2 == 0 else SLC1):
                    acc_lo = acc_lo + buf[s, 0:16]
                    acc_hi = acc_hi + buf[s, 16:32]
            row = ROWS_PER_G * g + r
            outs_v[row, 0:16] = acc_lo
            outs_v[row, 16:32] = acc_hi
        return carry

    lax.fori_loop(0, GROUPS, group, 0)
    pltpu.sync_copy(outs_v, out_hbm.at[pl.ds(base_row, ROWS_PER_W)])


_pooled_sum = functools.partial(
    pl.kernel,
    mesh=plsc.VectorSubcoreMesh(core_axis_name="c", subcore_axis_name="s"),
    compiler_params=pltpu.CompilerParams(use_tc_tiling_on_sc=False),
    out_type=jax.ShapeDtypeStruct((BATCH, EMB), jnp.float32),
    scratch_types=[
        pltpu.VMEM((ROWS_PER_W, SEQ), jnp.int32),
        pltpu.VMEM((SLC0, EMB), jnp.float32),
        pltpu.VMEM((SLC1, EMB), jnp.float32),
        pltpu.VMEM((SLC0, EMB), jnp.float32),
        pltpu.VMEM((SLC1, EMB), jnp.float32),
        pltpu.VMEM((SLC0, EMB), jnp.float32),
        pltpu.VMEM((SLC1, EMB), jnp.float32),
        pltpu.VMEM((SLC0, EMB), jnp.float32),
        pltpu.VMEM((SLC1, EMB), jnp.float32),
        pltpu.VMEM((SLC0, EMB), jnp.float32),
        pltpu.VMEM((SLC1, EMB), jnp.float32),
        pltpu.VMEM((SLC0, EMB), jnp.float32),
        pltpu.VMEM((SLC1, EMB), jnp.float32),
        pltpu.VMEM((SLC0, EMB), jnp.float32),
        pltpu.VMEM((SLC1, EMB), jnp.float32),
        pltpu.VMEM((SLC0, EMB), jnp.float32),
        pltpu.VMEM((SLC1, EMB), jnp.float32),
        pltpu.VMEM((ROWS_PER_W, EMB), jnp.float32),
        pltpu.SemaphoreType.DMA,
        pltpu.SemaphoreType.DMA,
        pltpu.SemaphoreType.DMA,
        pltpu.SemaphoreType.DMA,
        pltpu.SemaphoreType.DMA,
        pltpu.SemaphoreType.DMA,
        pltpu.SemaphoreType.DMA,
        pltpu.SemaphoreType.DMA,
        pltpu.SemaphoreType.DMA,
        pltpu.SemaphoreType.DMA,
        pltpu.SemaphoreType.DMA,
        pltpu.SemaphoreType.DMA,
        pltpu.SemaphoreType.DMA,
        pltpu.SemaphoreType.DMA,
        pltpu.SemaphoreType.DMA,
        pltpu.SemaphoreType.DMA,
    ],
)(_pool_body)


def _mlp_body(pooled_ref, idx_ref, t0_ref, w1_ref, bb1_ref, w2_ref, bb2_ref,
              out_ref):
    pooled = pooled_ref[...]
    idx = idx_ref[...]
    c0 = jnp.sum((idx == 0).astype(jnp.float32), axis=1, keepdims=True)
    x = (pooled - c0 * t0_ref[...]) * (1.0 / SEQ)
    h = jnp.dot(x, w1_ref[...], preferred_element_type=jnp.float32,
                precision=lax.Precision.HIGHEST) + bb1_ref[...]
    z = jnp.dot(h, w2_ref[...], preferred_element_type=jnp.float32,
                precision=lax.Precision.HIGHEST) + bb2_ref[...]
    z = z - jnp.max(z, axis=1, keepdims=True)
    e = jnp.exp(z)
    out_ref[...] = e / jnp.sum(e, axis=1, keepdims=True)


def _mlp_call(pooled, idx, t0, w1, bb1, w2, bb2):
    bt = 512
    grid = (BATCH // bt,)
    return pl.pallas_call(
        _mlp_body,
        out_shape=jax.ShapeDtypeStruct((BATCH, OUT), jnp.float32),
        grid=grid,
        in_specs=[
            pl.BlockSpec((bt, EMB), lambda i: (i, 0)),
            pl.BlockSpec((bt, SEQ), lambda i: (i, 0)),
            pl.BlockSpec((1, EMB), lambda i: (0, 0)),
            pl.BlockSpec((EMB, HID), lambda i: (0, 0)),
            pl.BlockSpec((1, HID), lambda i: (0, 0)),
            pl.BlockSpec((HID, OUT), lambda i: (0, 0)),
            pl.BlockSpec((1, OUT), lambda i: (0, 0)),
        ],
        out_specs=pl.BlockSpec((bt, OUT), lambda i: (i, 0)),
    )(pooled, idx, t0, w1, bb1, w2, bb2)


def kernel(inputs, table, W1, b1, W2, b2):
    idx = inputs.astype(jnp.int32)
    pooled = _pooled_sum(table, idx)
    t0 = table[0:1]
    return _mlp_call(pooled, idx, t0, W1, b1.reshape(1, HID), W2,
                     b2.reshape(1, OUT))
